# Initial kernel scaffold; baseline (speedup 1.0000x reference)
import jax
import jax.numpy as jnp
from jax.experimental import pallas as pl


def kernel(x, embeddings):
    raise NotImplementedError("write your pallas kernel here")


# speedup vs baseline: 20.2509x; 20.2509x over previous
import functools
import math

import jax
import jax.numpy as jnp
import numpy as np
from jax import lax
from jax._src import config as _jax_config
from jax.experimental import pallas as pl
from jax.experimental.pallas import tpu as pltpu
from jax.experimental.pallas import tpu_sc as plsc

L = 16
F = 2
T = 524288
N_MIN, N_MAX = 16, 2048
_b = math.exp((math.log(N_MAX) - math.log(N_MIN)) / (L - 1))
RESOLUTIONS = [math.floor(N_MIN * _b ** i) for i in range(L)]
P1D = np.int32(np.array((2 * 2654435761) % (1 << 32), np.uint64)
               .astype(np.uint32).view(np.int32))
P2D = np.int32(2 * 805459861)
MASKD = np.int32((T - 1) << 1)

N_PTS = 262144
NC, NS = 2, 16
NW = NC * NS
PTS_PER_W = N_PTS // NW
GRP = 16
BLK = 512
NG = BLK // GRP
NBLK = PTS_PER_W // BLK
NCORNER = 8


def _encode_kernel(xf_hbm, emb_hbm, out_hbm, x_v, out_v, idx_v, rows_v, sem):
    wid = (lax.axis_index("s").astype(jnp.int32) * jnp.int32(NC)
           + lax.axis_index("c").astype(jnp.int32))
    pbase = wid * jnp.int32(PTS_PER_W)
    lanes = lax.iota(jnp.int32, GRP)
    lanes3 = lanes * jnp.int32(3)

    def _block(blk, _):
        row0 = pbase + blk * jnp.int32(BLK)
        pltpu.sync_copy(xf_hbm.at[pl.ds(row0 * jnp.int32(3), BLK * 3)], x_v)

        def _group(g, _):
            goff3 = g * jnp.int32(GRP * 3)
            px = plsc.load_gather(x_v, [goff3 + lanes3])
            py = plsc.load_gather(x_v, [goff3 + lanes3 + jnp.int32(1)])
            pz = plsc.load_gather(x_v, [goff3 + lanes3 + jnp.int32(2)])

            for i, res in enumerate(RESOLUTIONS):
                resf = jnp.float32(res)
                ix = (px * resf).astype(jnp.int32)
                iy = (py * resf).astype(jnp.int32)
                iz = (pz * resf).astype(jnp.int32)
                hx0 = ix + ix
                hx1 = hx0 + jnp.int32(2)
                hy0 = iy * P1D
                hy1 = hy0 + P1D
                hz0 = iz * P2D
                hz1 = hz0 + P2D
                lvl = jnp.int32(i * T * F)
                for c in range(NCORNER):
                    hx = hx1 if (c & 4) else hx0
                    hy = hy1 if (c & 2) else hy0
                    hz = hz1 if (c & 1) else hz0
                    w0 = ((hx ^ hy ^ hz) & MASKD) + lvl
                    idx_v[i, 0, pl.ds(c * GRP, GRP)] = w0
                    idx_v[i, 1, pl.ds(c * GRP, GRP)] = w0 + jnp.int32(1)

            for i in range(L):
                for f in range(F):
                    pltpu.async_copy(emb_hbm.at[idx_v.at[i, f]],
                                     rows_v.at[i, f], sem)
            for i in range(L):
                for f in range(F):
                    pltpu.make_async_copy(emb_hbm.at[idx_v.at[i, f]],
                                          rows_v.at[i, f], sem).wait()

            out_rows = g * jnp.int32(GRP) + lanes
            for i, res in enumerate(RESOLUTIONS):
                resf = jnp.float32(res)
                xs = px * resf
                ys = py * resf
                zs = pz * resf
                fx = xs - xs.astype(jnp.int32).astype(jnp.float32)
                fy = ys - ys.astype(jnp.int32).astype(jnp.float32)
                fz = zs - zs.astype(jnp.int32).astype(jnp.float32)
                for f in range(F):
                    v = [rows_v[i, f, pl.ds(c * GRP, GRP)]
                         for c in range(NCORNER)]
                    c00 = v[0] + (v[4] - v[0]) * fx
                    c01 = v[1] + (v[5] - v[1]) * fx
                    c10 = v[2] + (v[6] - v[2]) * fx
                    c11 = v[3] + (v[7] - v[3]) * fx
                    c0 = c00 + (c10 - c00) * fy
                    c1 = c01 + (c11 - c01) * fy
                    cc = c0 + (c1 - c0) * fz
                    plsc.store_scatter(
                        out_v, [out_rows, jnp.full((GRP,), i * F + f,
                                                   jnp.int32)], cc)
            return _

        lax.fori_loop(np.int32(0), np.int32(NG), _group, None)
        pltpu.sync_copy(out_v, out_hbm.at[pl.ds(row0, BLK)])
        return _

    lax.fori_loop(np.int32(0), np.int32(NBLK), _block, None)


@jax.jit
def _encode(xf, emb):
    call = pl.kernel(
        _encode_kernel,
        out_type=jax.ShapeDtypeStruct((N_PTS, L * F), jnp.float32),
        mesh=plsc.VectorSubcoreMesh(core_axis_name="c", subcore_axis_name="s",
                                    num_cores=NC, num_subcores=NS),
        scratch_types=[
            pltpu.VMEM((BLK * 3,), jnp.float32),
            pltpu.VMEM((BLK, L * F), jnp.float32),
            pltpu.VMEM((L, F, NCORNER * GRP), jnp.int32),
            pltpu.VMEM((L, F, NCORNER * GRP), jnp.float32),
            pltpu.SemaphoreType.DMA,
        ],
        compiler_params=pltpu.CompilerParams(needs_layout_passes=False),
    )
    return call(xf, emb)


def kernel(x, embeddings):
    xf = x.astype(jnp.float32).reshape(N_PTS * 3)
    emb = embeddings.astype(jnp.float32).reshape(L * T * F)
    with _jax_config.enable_x64(False):
        return _encode(xf, emb)


# speedup vs baseline: 21.4078x; 1.0571x over previous
import math

import jax
import jax.numpy as jnp
import numpy as np
from jax import lax
from jax._src import config as _jax_config
from jax.experimental import pallas as pl
from jax.experimental.pallas import tpu as pltpu
from jax.experimental.pallas import tpu_sc as plsc

L = 16
F = 2
T = 524288
N_MIN, N_MAX = 16, 2048
_b = math.exp((math.log(N_MAX) - math.log(N_MIN)) / (L - 1))
RESOLUTIONS = [math.floor(N_MIN * _b ** i) for i in range(L)]
P1D = np.int32(np.array((2 * 2654435761) % (1 << 32), np.uint64)
               .astype(np.uint32).view(np.int32))
P2D = np.int32(2 * 805459861)
MASKD = np.int32((T - 1) << 1)

N_PTS = 262144
NC, NS = 2, 16
NW = NC * NS
PTS_PER_W = N_PTS // NW
GRP = 16
BLK = 512
NG = BLK // GRP
NBLK = PTS_PER_W // BLK
NCORNER = 8


def _encode_kernel(xt_hbm, emb_hbm, out_hbm, x_v, out_v,
                   idx0_v, idx1_v, rows0_v, rows1_v, sem0, sem1):
    wid = (lax.axis_index("s").astype(jnp.int32) * jnp.int32(NC)
           + lax.axis_index("c").astype(jnp.int32))
    pbase = wid * jnp.int32(PTS_PER_W)
    lanes = lax.iota(jnp.int32, GRP)

    def _compute_idx(g, idx_v):
        goff = g * jnp.int32(GRP)
        px = x_v[pl.ds(goff, GRP)]
        py = x_v[pl.ds(jnp.int32(BLK) + goff, GRP)]
        pz = x_v[pl.ds(jnp.int32(2 * BLK) + goff, GRP)]
        for i, res in enumerate(RESOLUTIONS):
            resf = jnp.float32(res)
            ix = (px * resf).astype(jnp.int32)
            iy = (py * resf).astype(jnp.int32)
            iz = (pz * resf).astype(jnp.int32)
            hx0 = ix + ix
            hx1 = hx0 + jnp.int32(2)
            hy0 = iy * P1D
            hy1 = hy0 + P1D
            hz0 = iz * P2D
            hz1 = hz0 + P2D
            lvl = jnp.int32(i * T * F)
            for c in range(NCORNER):
                hx = hx1 if (c & 4) else hx0
                hy = hy1 if (c & 2) else hy0
                hz = hz1 if (c & 1) else hz0
                w0 = ((hx ^ hy ^ hz) & MASKD) + lvl
                idx_v[i, 0, pl.ds(c * GRP, GRP)] = w0
                idx_v[i, 1, pl.ds(c * GRP, GRP)] = w0 + jnp.int32(1)

    def _fire(idx_v, rows_v, sem):
        for i in range(L):
            for f in range(F):
                pltpu.async_copy(emb_hbm.at[idx_v.at[i, f]],
                                 rows_v.at[i, f], sem)

    def _drain(idx_v, rows_v, sem):
        for i in range(L):
            for f in range(F):
                pltpu.make_async_copy(emb_hbm.at[idx_v.at[i, f]],
                                      rows_v.at[i, f], sem).wait()

    def _interp(g, rows_v):
        goff = g * jnp.int32(GRP)
        px = x_v[pl.ds(goff, GRP)]
        py = x_v[pl.ds(jnp.int32(BLK) + goff, GRP)]
        pz = x_v[pl.ds(jnp.int32(2 * BLK) + goff, GRP)]
        out_rows = goff + lanes
        for i, res in enumerate(RESOLUTIONS):
            resf = jnp.float32(res)
            xs = px * resf
            ys = py * resf
            zs = pz * resf
            fx = xs - xs.astype(jnp.int32).astype(jnp.float32)
            fy = ys - ys.astype(jnp.int32).astype(jnp.float32)
            fz = zs - zs.astype(jnp.int32).astype(jnp.float32)
            for f in range(F):
                v = [rows_v[i, f, pl.ds(c * GRP, GRP)]
                     for c in range(NCORNER)]
                c00 = v[0] + (v[4] - v[0]) * fx
                c01 = v[1] + (v[5] - v[1]) * fx
                c10 = v[2] + (v[6] - v[2]) * fx
                c11 = v[3] + (v[7] - v[3]) * fx
                c0 = c00 + (c10 - c00) * fy
                c1 = c01 + (c11 - c01) * fy
                cc = c0 + (c1 - c0) * fz
                plsc.store_scatter(
                    out_v, [out_rows, jnp.full((GRP,), i * F + f,
                                               jnp.int32)], cc)

    def _block(blk, _):
        row0 = pbase + blk * jnp.int32(BLK)
        for d in range(3):
            pltpu.sync_copy(
                xt_hbm.at[pl.ds(jnp.int32(d * N_PTS) + row0, BLK)],
                x_v.at[pl.ds(d * BLK, BLK)])

        _compute_idx(jnp.int32(0), idx0_v)
        _fire(idx0_v, rows0_v, sem0)

        def _pair(it, _):
            g0 = it * jnp.int32(2)
            g1 = g0 + jnp.int32(1)
            _compute_idx(g1, idx1_v)
            _fire(idx1_v, rows1_v, sem1)
            _drain(idx0_v, rows0_v, sem0)
            _interp(g0, rows0_v)

            @pl.when(it < jnp.int32(NG // 2 - 1))
            def _tail():
                _compute_idx(g1 + jnp.int32(1), idx0_v)
                _fire(idx0_v, rows0_v, sem0)

            _drain(idx1_v, rows1_v, sem1)
            _interp(g1, rows1_v)
            return _

        lax.fori_loop(np.int32(0), np.int32(NG // 2), _pair, None)
        pltpu.sync_copy(out_v, out_hbm.at[pl.ds(row0, BLK)])
        return _

    lax.fori_loop(np.int32(0), np.int32(NBLK), _block, None)


@jax.jit
def _encode(xt, emb):
    call = pl.kernel(
        _encode_kernel,
        out_type=jax.ShapeDtypeStruct((N_PTS, L * F), jnp.float32),
        mesh=plsc.VectorSubcoreMesh(core_axis_name="c", subcore_axis_name="s",
                                    num_cores=NC, num_subcores=NS),
        scratch_types=[
            pltpu.VMEM((3 * BLK,), jnp.float32),
            pltpu.VMEM((BLK, L * F), jnp.float32),
            pltpu.VMEM((L, F, NCORNER * GRP), jnp.int32),
            pltpu.VMEM((L, F, NCORNER * GRP), jnp.int32),
            pltpu.VMEM((L, F, NCORNER * GRP), jnp.float32),
            pltpu.VMEM((L, F, NCORNER * GRP), jnp.float32),
            pltpu.SemaphoreType.DMA,
            pltpu.SemaphoreType.DMA,
        ],
        compiler_params=pltpu.CompilerParams(needs_layout_passes=False),
    )
    return call(xt, emb)


def kernel(x, embeddings):
    xt = x.astype(jnp.float32).T.reshape(3 * N_PTS)
    emb = embeddings.astype(jnp.float32).reshape(L * T * F)
    with _jax_config.enable_x64(False):
        return _encode(xt, emb)
